# Initial kernel scaffold; baseline (speedup 1.0000x reference)
import jax
import jax.numpy as jnp
from jax.experimental import pallas as pl


def kernel(x, edge_index, W1, b1, W2, b2):
    raise NotImplementedError("write your pallas kernel here")


# speedup vs baseline: 17.1149x; 17.1149x over previous
import functools

import jax
import jax.numpy as jnp
from jax import lax
from jax.experimental import pallas as pl
from jax.experimental.pallas import tpu as pltpu
from jax.experimental.pallas import tpu_sc as plsc

N = 10000
IN_DIM = 128
HID = 64
OUT = 3
OUTP = 16

NC = 2
NS = 16
NW = NC * NS
CHUNK = 128
N_ACC = 10240
ROWS_PER_TILE = N_ACC // NS
DEGW = 16

E = 320000
NCHUNK = -(-E // (NW * CHUNK))
E_PAD = NW * NCHUNK * CHUNK

ROWBLK = 1000


def _zero_vmem(buf, rows, width):
    z16 = jnp.zeros((16,), jnp.float32)
    per_row = width // 16

    def body(i, _):
        r = i // per_row
        c = (i % per_row) * 16
        buf[r, pl.ds(c, 16)] = z16
        return 0

    lax.fori_loop(0, rows * per_row, body, 0)


def _make_sc_agg(width, nchunk):
    mesh = plsc.VectorSubcoreMesh(core_axis_name="c", subcore_axis_name="s", num_cores=NC, num_subcores=NS)
    zrows = 64

    @functools.partial(
        pl.kernel,
        out_type=jax.ShapeDtypeStruct((NC, N_ACC, width), jnp.float32),
        mesh=mesh,
        compiler_params=pltpu.CompilerParams(use_tc_tiling_on_sc=False),
        scratch_types=[
            pltpu.VMEM((CHUNK,), jnp.int32),
            pltpu.VMEM((CHUNK,), jnp.int32),
            pltpu.VMEM((CHUNK, width), jnp.float32),
            pltpu.VMEM((zrows, width), jnp.float32),
            pltpu.VMEM_SHARED((N_ACC, width), jnp.float32),
            pltpu.SemaphoreType.DMA,
        ],
    )
    def agg(g_hbm, src_hbm, dst_hbm, out_hbm, sidx, didx, rows, zbuf, acc, sem):
        c = lax.axis_index("c")
        s = lax.axis_index("s")
        wid = s * NC + c
        _zero_vmem(zbuf, zrows, width)
        row0 = s * ROWS_PER_TILE
        for t in range(ROWS_PER_TILE // zrows):
            pltpu.sync_copy(zbuf, acc.at[pl.ds(row0 + t * zrows, zrows)])
        plsc.subcore_barrier()

        def body(k, _):
            base = pl.multiple_of((wid * nchunk + k) * CHUNK, 8)
            pltpu.sync_copy(src_hbm.at[pl.ds(base, CHUNK)], sidx)
            pltpu.sync_copy(dst_hbm.at[pl.ds(base, CHUNK)], didx)
            pltpu.async_copy(g_hbm.at[sidx], rows, sem).wait()
            pltpu.sync_copy(rows, acc.at[didx], add=True)
            return 0

        lax.fori_loop(0, nchunk, body, 0)
        plsc.subcore_barrier()
        pltpu.sync_copy(acc.at[pl.ds(row0, ROWS_PER_TILE)],
                        out_hbm.at[c, pl.ds(row0, ROWS_PER_TILE)])

    return agg


def _make_sc_degree(nchunk):
    mesh = plsc.VectorSubcoreMesh(core_axis_name="c", subcore_axis_name="s", num_cores=NC, num_subcores=NS)
    zrows = 64

    @functools.partial(
        pl.kernel,
        out_type=jax.ShapeDtypeStruct((NC, N_ACC, DEGW), jnp.float32),
        mesh=mesh,
        compiler_params=pltpu.CompilerParams(use_tc_tiling_on_sc=False),
        scratch_types=[
            pltpu.VMEM((CHUNK,), jnp.int32),
            pltpu.VMEM((CHUNK, DEGW), jnp.float32),
            pltpu.VMEM((zrows, DEGW), jnp.float32),
            pltpu.VMEM_SHARED((N_ACC, DEGW), jnp.float32),
        ],
    )
    def deg(dst_hbm, out_hbm, didx, ones, zbuf, acc):
        c = lax.axis_index("c")
        s = lax.axis_index("s")
        wid = s * NC + c
        _zero_vmem(zbuf, zrows, DEGW)
        one16 = jnp.ones((16,), jnp.float32)

        def fill(i, _):
            ones[i, pl.ds(0, 16)] = one16
            return 0

        lax.fori_loop(0, CHUNK, fill, 0)
        row0 = s * ROWS_PER_TILE
        for t in range(ROWS_PER_TILE // zrows):
            pltpu.sync_copy(zbuf, acc.at[pl.ds(row0 + t * zrows, zrows)])
        plsc.subcore_barrier()

        def body(k, _):
            base = pl.multiple_of((wid * nchunk + k) * CHUNK, 8)
            pltpu.sync_copy(dst_hbm.at[pl.ds(base, CHUNK)], didx)
            pltpu.sync_copy(ones, acc.at[didx], add=True)
            return 0

        lax.fori_loop(0, nchunk, body, 0)
        plsc.subcore_barrier()
        pltpu.sync_copy(acc.at[pl.ds(row0, ROWS_PER_TILE)],
                        out_hbm.at[c, pl.ds(row0, ROWS_PER_TILE)])

    return deg


def _dinv_from_deg(d_blk):
    deg = 1.0 + d_blk[0] + d_blk[1]
    return lax.rsqrt(deg)[:, 0:1]


def _tc_g1(x, W1, degp):
    def body(x_ref, w_ref, d_ref, o_ref):
        dinv = _dinv_from_deg(d_ref[...])
        h = jnp.dot(x_ref[...], w_ref[...], preferred_element_type=jnp.float32)
        o_ref[...] = h * dinv

    return pl.pallas_call(
        body,
        grid=(N // ROWBLK,),
        in_specs=[
            pl.BlockSpec((ROWBLK, IN_DIM), lambda i: (i, 0)),
            pl.BlockSpec((IN_DIM, HID), lambda i: (0, 0)),
            pl.BlockSpec((NC, ROWBLK, DEGW), lambda i: (0, i, 0)),
        ],
        out_specs=pl.BlockSpec((ROWBLK, HID), lambda i: (i, 0)),
        out_shape=jax.ShapeDtypeStruct((N, HID), jnp.float32),
    )(x, W1, degp)


def _tc_mid(aggp, g1, degp, b1, W2p):
    def body(a_ref, g_ref, d_ref, b_ref, w_ref, o_ref):
        dinv = _dinv_from_deg(d_ref[...])
        tot = a_ref[0] + a_ref[1] + g_ref[...]
        z = jnp.maximum(tot * dinv + b_ref[...], 0.0)
        h2 = jnp.dot(z, w_ref[...], preferred_element_type=jnp.float32)
        o_ref[...] = h2 * dinv

    return pl.pallas_call(
        body,
        grid=(N // ROWBLK,),
        in_specs=[
            pl.BlockSpec((NC, ROWBLK, HID), lambda i: (0, i, 0)),
            pl.BlockSpec((ROWBLK, HID), lambda i: (i, 0)),
            pl.BlockSpec((NC, ROWBLK, DEGW), lambda i: (0, i, 0)),
            pl.BlockSpec((1, HID), lambda i: (0, 0)),
            pl.BlockSpec((HID, OUTP), lambda i: (0, 0)),
        ],
        out_specs=pl.BlockSpec((ROWBLK, OUTP), lambda i: (i, 0)),
        out_shape=jax.ShapeDtypeStruct((N, OUTP), jnp.float32),
    )(aggp, g1, degp, b1, W2p)


def _tc_final(aggp, g2, degp, b2p):
    def body(a_ref, g_ref, d_ref, b_ref, o_ref):
        dinv = _dinv_from_deg(d_ref[...])
        o = (a_ref[0] + a_ref[1] + g_ref[...]) * dinv + b_ref[...]
        col = lax.broadcasted_iota(jnp.int32, (ROWBLK, OUTP), 1)
        valid = col < OUT
        neg = jnp.where(valid, o, -jnp.inf)
        mx = jnp.max(neg, axis=1, keepdims=True)
        ssum = jnp.sum(jnp.where(valid, jnp.exp(o - mx), 0.0), axis=1,
                       keepdims=True)
        res = o - (jnp.log(ssum) + mx)
        o_ref[...] = res[:, :OUT]

    return pl.pallas_call(
        body,
        grid=(N // ROWBLK,),
        in_specs=[
            pl.BlockSpec((NC, ROWBLK, OUTP), lambda i: (0, i, 0)),
            pl.BlockSpec((ROWBLK, OUTP), lambda i: (i, 0)),
            pl.BlockSpec((NC, ROWBLK, DEGW), lambda i: (0, i, 0)),
            pl.BlockSpec((1, OUTP), lambda i: (0, 0)),
        ],
        out_specs=pl.BlockSpec((ROWBLK, OUT), lambda i: (i, 0)),
        out_shape=jax.ShapeDtypeStruct((N, OUT), jnp.float32),
    )(aggp, g2, degp, b2p)


@functools.lru_cache(maxsize=None)
def _get_sc_kernels():
    return (_make_sc_degree(NCHUNK),
            _make_sc_agg(HID, NCHUNK),
            _make_sc_agg(OUTP, NCHUNK))


def kernel(x, edge_index, W1, b1, W2, b2):
    src = edge_index[0].astype(jnp.int32)
    dst = edge_index[1].astype(jnp.int32)
    npad = E_PAD - E
    src_p = jnp.concatenate([src, jnp.zeros((npad,), jnp.int32)])
    dst_p = jnp.concatenate(
        [dst, N + (jnp.arange(npad, dtype=jnp.int32) % (N_ACC - N))])

    W2p = jnp.pad(W2, ((0, 0), (0, OUTP - OUT)))
    b1r = b1.reshape(1, HID)
    b2p = jnp.pad(b2, (0, OUTP - OUT)).reshape(1, OUTP)

    sc_degree, sc_agg64, sc_agg16 = _get_sc_kernels()
    degp = sc_degree(dst_p)
    g1 = _tc_g1(x, W1, degp)
    aggp1 = sc_agg64(g1, src_p, dst_p)
    g2 = _tc_mid(aggp1, g1, degp, b1r, W2p)
    aggp2 = sc_agg16(g2, src_p, dst_p)
    return _tc_final(aggp2, g2, degp, b2p)


# speedup vs baseline: 22.0023x; 1.2856x over previous
import functools

import jax
import jax.numpy as jnp
from jax import lax
from jax.experimental import pallas as pl
from jax.experimental.pallas import tpu as pltpu
from jax.experimental.pallas import tpu_sc as plsc

N = 10000
IN_DIM = 128
HID = 64
OUT = 3
OUTP = 16

NC = 2
NS = 16
NW = NC * NS
CHUNK = 128
N_ACC = 10240
ROWS_PER_TILE = N_ACC // NS
DEGW = 16

E = 320000
SUP = 4
NSUP = 20
NCHUNK = NSUP * SUP
E_PAD = NW * NCHUNK * CHUNK

ROWBLK = 1000


def _zero_vmem(buf, rows, width):
    z16 = jnp.zeros((16,), jnp.float32)
    per_row = width // 16

    def body(i, _):
        r = i // per_row
        c = (i % per_row) * 16
        buf[r, pl.ds(c, 16)] = z16
        return 0

    lax.fori_loop(0, rows * per_row, body, 0)


def _make_sc_agg(width):
    mesh = plsc.VectorSubcoreMesh(core_axis_name="c", subcore_axis_name="s",
                                  num_cores=NC, num_subcores=NS)
    zrows = 64

    @functools.partial(
        pl.kernel,
        out_type=jax.ShapeDtypeStruct((NC, N_ACC, width), jnp.float32),
        mesh=mesh,
        compiler_params=pltpu.CompilerParams(use_tc_tiling_on_sc=False),
        scratch_types=[
            pltpu.VMEM((SUP, CHUNK), jnp.int32),
            pltpu.VMEM((SUP, CHUNK), jnp.int32),
            pltpu.VMEM((SUP, CHUNK), jnp.int32),
            pltpu.VMEM((SUP, CHUNK), jnp.int32),
            pltpu.VMEM((SUP, CHUNK, width), jnp.float32),
            pltpu.VMEM((SUP, CHUNK, width), jnp.float32),
            pltpu.VMEM((zrows, width), jnp.float32),
            pltpu.VMEM_SHARED((N_ACC, width), jnp.float32),
            pltpu.SemaphoreType.DMA,
            pltpu.SemaphoreType.DMA,
            pltpu.SemaphoreType.DMA,
            pltpu.SemaphoreType.DMA,
        ],
    )
    def agg(g_hbm, src_hbm, dst_hbm, out_hbm, sidx0, sidx1, didx0, didx1,
            rows0, rows1, zbuf, acc, semi0, semi1, semg, sems):
        c = lax.axis_index("c")
        s = lax.axis_index("s")
        wid = s * NC + c
        bufs = ((sidx0, didx0, rows0, semi0), (sidx1, didx1, rows1, semi1))
        _zero_vmem(zbuf, zrows, width)
        row0 = s * ROWS_PER_TILE
        for t in range(ROWS_PER_TILE // zrows):
            pltpu.sync_copy(zbuf, acc.at[pl.ds(row0 + t * zrows, zrows)])
        plsc.subcore_barrier()

        def issue_idx(k, b):
            sidx, didx, _, semi = bufs[b]
            crow = (wid * NSUP + k) * SUP
            pltpu.async_copy(src_hbm.at[pl.ds(crow, SUP)], sidx, semi)
            pltpu.async_copy(dst_hbm.at[pl.ds(crow, SUP)], didx, semi)

        def one_super(k, b, prefetch):
            sidx, didx, rows, semi = bufs[b]
            pltpu.make_async_copy(src_hbm.at[pl.ds(0, SUP)], sidx, semi).wait()
            pltpu.make_async_copy(dst_hbm.at[pl.ds(0, SUP)], didx, semi).wait()
            if prefetch:
                issue_idx(k + 1, b ^ 1)
            for j in range(SUP):
                pltpu.async_copy(g_hbm.at[sidx.at[j]], rows.at[j], semg)
            for j in range(SUP):
                pltpu.make_async_copy(g_hbm.at[sidx.at[j]], rows.at[j],
                                      semg).wait()
            for j in range(SUP):
                pltpu.async_copy(rows.at[j], acc.at[didx.at[j]], sems,
                                 add=True)
            for j in range(SUP):
                pltpu.make_async_copy(rows.at[j], acc.at[didx.at[j]],
                                      sems).wait()

        issue_idx(0, 0)

        def block(i, _):
            one_super(2 * i, 0, True)
            one_super(2 * i + 1, 1, True)
            return 0

        lax.fori_loop(0, NSUP // 2 - 1, block, 0)
        one_super(NSUP - 2, 0, True)
        one_super(NSUP - 1, 1, False)

        plsc.subcore_barrier()
        pltpu.sync_copy(acc.at[pl.ds(row0, ROWS_PER_TILE)],
                        out_hbm.at[c, pl.ds(row0, ROWS_PER_TILE)])

    return agg


def _make_sc_degree():
    mesh = plsc.VectorSubcoreMesh(core_axis_name="c", subcore_axis_name="s",
                                  num_cores=NC, num_subcores=NS)
    zrows = 64

    @functools.partial(
        pl.kernel,
        out_type=jax.ShapeDtypeStruct((NC, N_ACC, DEGW), jnp.float32),
        mesh=mesh,
        compiler_params=pltpu.CompilerParams(use_tc_tiling_on_sc=False),
        scratch_types=[
            pltpu.VMEM((SUP, CHUNK), jnp.int32),
            pltpu.VMEM((SUP, CHUNK), jnp.int32),
            pltpu.VMEM((CHUNK, DEGW), jnp.float32),
            pltpu.VMEM((zrows, DEGW), jnp.float32),
            pltpu.VMEM_SHARED((N_ACC, DEGW), jnp.float32),
            pltpu.SemaphoreType.DMA,
            pltpu.SemaphoreType.DMA,
            pltpu.SemaphoreType.DMA,
        ],
    )
    def deg(dst_hbm, out_hbm, didx0, didx1, ones, zbuf, acc, semi0, semi1,
            sems):
        c = lax.axis_index("c")
        s = lax.axis_index("s")
        wid = s * NC + c
        bufs = ((didx0, semi0), (didx1, semi1))
        _zero_vmem(zbuf, zrows, DEGW)
        one16 = jnp.ones((16,), jnp.float32)

        def fill(i, _):
            ones[i, pl.ds(0, 16)] = one16
            return 0

        lax.fori_loop(0, CHUNK, fill, 0)
        row0 = s * ROWS_PER_TILE
        for t in range(ROWS_PER_TILE // zrows):
            pltpu.sync_copy(zbuf, acc.at[pl.ds(row0 + t * zrows, zrows)])
        plsc.subcore_barrier()

        def issue_idx(k, b):
            didx, semi = bufs[b]
            crow = (wid * NSUP + k) * SUP
            pltpu.async_copy(dst_hbm.at[pl.ds(crow, SUP)], didx, semi)

        def one_super(k, b, prefetch):
            didx, semi = bufs[b]
            pltpu.make_async_copy(dst_hbm.at[pl.ds(0, SUP)], didx, semi).wait()
            if prefetch:
                issue_idx(k + 1, b ^ 1)
            for j in range(SUP):
                pltpu.async_copy(ones, acc.at[didx.at[j]], sems, add=True)
            for j in range(SUP):
                pltpu.make_async_copy(ones, acc.at[didx.at[j]], sems).wait()

        issue_idx(0, 0)

        def block(i, _):
            one_super(2 * i, 0, True)
            one_super(2 * i + 1, 1, True)
            return 0

        lax.fori_loop(0, NSUP // 2 - 1, block, 0)
        one_super(NSUP - 2, 0, True)
        one_super(NSUP - 1, 1, False)

        plsc.subcore_barrier()
        pltpu.sync_copy(acc.at[pl.ds(row0, ROWS_PER_TILE)],
                        out_hbm.at[c, pl.ds(row0, ROWS_PER_TILE)])

    return deg


def _dinv_from_deg(d_blk):
    deg = 1.0 + d_blk[0] + d_blk[1]
    return lax.rsqrt(deg)[:, 0:1]


def _tc_g1(x, W1, degp):
    def body(x_ref, w_ref, d_ref, o_ref):
        dinv = _dinv_from_deg(d_ref[...])
        h = jnp.dot(x_ref[...], w_ref[...], preferred_element_type=jnp.float32)
        o_ref[...] = h * dinv

    return pl.pallas_call(
        body,
        grid=(N // ROWBLK,),
        in_specs=[
            pl.BlockSpec((ROWBLK, IN_DIM), lambda i: (i, 0)),
            pl.BlockSpec((IN_DIM, HID), lambda i: (0, 0)),
            pl.BlockSpec((NC, ROWBLK, DEGW), lambda i: (0, i, 0)),
        ],
        out_specs=pl.BlockSpec((ROWBLK, HID), lambda i: (i, 0)),
        out_shape=jax.ShapeDtypeStruct((N, HID), jnp.float32),
    )(x, W1, degp)


def _tc_mid(aggp, g1, degp, b1, W2p):
    def body(a_ref, g_ref, d_ref, b_ref, w_ref, o_ref):
        dinv = _dinv_from_deg(d_ref[...])
        tot = a_ref[0] + a_ref[1] + g_ref[...]
        z = jnp.maximum(tot * dinv + b_ref[...], 0.0)
        h2 = jnp.dot(z, w_ref[...], preferred_element_type=jnp.float32)
        o_ref[...] = h2 * dinv

    return pl.pallas_call(
        body,
        grid=(N // ROWBLK,),
        in_specs=[
            pl.BlockSpec((NC, ROWBLK, HID), lambda i: (0, i, 0)),
            pl.BlockSpec((ROWBLK, HID), lambda i: (i, 0)),
            pl.BlockSpec((NC, ROWBLK, DEGW), lambda i: (0, i, 0)),
            pl.BlockSpec((1, HID), lambda i: (0, 0)),
            pl.BlockSpec((HID, OUTP), lambda i: (0, 0)),
        ],
        out_specs=pl.BlockSpec((ROWBLK, OUTP), lambda i: (i, 0)),
        out_shape=jax.ShapeDtypeStruct((N, OUTP), jnp.float32),
    )(aggp, g1, degp, b1, W2p)


def _tc_final(aggp, g2, degp, b2p):
    def body(a_ref, g_ref, d_ref, b_ref, o_ref):
        dinv = _dinv_from_deg(d_ref[...])
        o = (a_ref[0] + a_ref[1] + g_ref[...]) * dinv + b_ref[...]
        col = lax.broadcasted_iota(jnp.int32, (ROWBLK, OUTP), 1)
        valid = col < OUT
        neg = jnp.where(valid, o, -jnp.inf)
        mx = jnp.max(neg, axis=1, keepdims=True)
        ssum = jnp.sum(jnp.where(valid, jnp.exp(o - mx), 0.0), axis=1,
                       keepdims=True)
        res = o - (jnp.log(ssum) + mx)
        o_ref[...] = res[:, :OUT]

    return pl.pallas_call(
        body,
        grid=(N // ROWBLK,),
        in_specs=[
            pl.BlockSpec((NC, ROWBLK, OUTP), lambda i: (0, i, 0)),
            pl.BlockSpec((ROWBLK, OUTP), lambda i: (i, 0)),
            pl.BlockSpec((NC, ROWBLK, DEGW), lambda i: (0, i, 0)),
            pl.BlockSpec((1, OUTP), lambda i: (0, 0)),
        ],
        out_specs=pl.BlockSpec((ROWBLK, OUT), lambda i: (i, 0)),
        out_shape=jax.ShapeDtypeStruct((N, OUT), jnp.float32),
    )(aggp, g2, degp, b2p)


@functools.lru_cache(maxsize=None)
def _get_sc_kernels():
    return (_make_sc_degree(),
            _make_sc_agg(HID),
            _make_sc_agg(OUTP))


def kernel(x, edge_index, W1, b1, W2, b2):
    src = edge_index[0].astype(jnp.int32)
    dst = edge_index[1].astype(jnp.int32)
    npad = E_PAD - E
    src_p = jnp.concatenate(
        [src, jnp.zeros((npad,), jnp.int32)]).reshape(-1, CHUNK)
    dst_p = jnp.concatenate(
        [dst, N + (jnp.arange(npad, dtype=jnp.int32) % (N_ACC - N))]
    ).reshape(-1, CHUNK)

    W2p = jnp.pad(W2, ((0, 0), (0, OUTP - OUT)))
    b1r = b1.reshape(1, HID)
    b2p = jnp.pad(b2, (0, OUTP - OUT)).reshape(1, OUTP)

    sc_degree, sc_agg64, sc_agg16 = _get_sc_kernels()
    degp = sc_degree(dst_p)
    g1 = _tc_g1(x, W1, degp)
    aggp1 = sc_agg64(g1, src_p, dst_p)
    g2 = _tc_mid(aggp1, g1, degp, b1r, W2p)
    aggp2 = sc_agg16(g2, src_p, dst_p)
    return _tc_final(aggp2, g2, degp, b2p)
